# Initial kernel scaffold; baseline (speedup 1.0000x reference)
import jax
import jax.numpy as jnp
from jax.experimental import pallas as pl


def kernel(ap_hid, ue_hid, ue2ap_hid, ap2ue_hid, W1, b1, W2, b2, W3, b3):
    raise NotImplementedError("write your pallas kernel here")


# speedup vs baseline: 1.4230x; 1.4230x over previous
import functools

import jax
import jax.numpy as jnp
from jax.experimental import pallas as pl


_HI = jax.lax.Precision.HIGHEST


def _pass1_body(n_ap, d, bu, nu,
                ap_ref, ue_ref, e1_ref, e2_ref, w1_ref, b1_ref, w2_ref,
                b2_ref, w3_ref, part_ref, uesum_ref, apsum_ref):
    j = pl.program_id(0)
    w1a = w1_ref[:d, :]
    w1e = w1_ref[d:, :]
    w2u = w2_ref[:d, :]
    w2e = w2_ref[d:, :]
    w3e = w3_ref[:d, :]
    w3g = w3_ref[d:, :]

    ap_pre = jnp.dot(ap_ref[...], w1a, precision=_HI,
                     preferred_element_type=jnp.float32) + b1_ref[...]
    ue_pre = jnp.dot(ue_ref[...], w2u, precision=_HI,
                     preferred_element_type=jnp.float32) + b2_ref[...]

    e1 = e1_ref[...].reshape(n_ap * bu, d)
    e2 = e2_ref[...].reshape(n_ap * bu, d)
    t1 = jnp.dot(e1, w1e, preferred_element_type=jnp.float32)
    t2 = jnp.dot(e2, w2e, preferred_element_type=jnp.float32)
    r1 = jax.nn.relu(t1.reshape(n_ap, bu, d) + ap_pre[:, None, :])
    r2 = jax.nn.relu(t2.reshape(n_ap, bu, d) + ue_pre[None, :, :])
    s = (r1 + r2).reshape(n_ap * bu, d)

    part = (jnp.dot(e2, w3e, preferred_element_type=jnp.float32)
            - jnp.dot(s, w3g, preferred_element_type=jnp.float32))
    part_ref[...] = part.reshape(n_ap, bu, d).astype(jnp.bfloat16)

    uesum_ref[...] = jnp.sum(r2, axis=0)

    @pl.when(j == 0)
    def _init():
        apsum_ref[...] = jnp.zeros_like(apsum_ref)

    apsum_ref[...] += jnp.sum(r1, axis=1)


def _pass2_body(d, part_ref, apsum_ref, uesum_ref, w3_ref, b3_ref, out_ref):
    w3g = w3_ref[d:, :]
    ap_add = jnp.dot(apsum_ref[...], w3g, precision=_HI,
                     preferred_element_type=jnp.float32)
    ue_add = jnp.dot(uesum_ref[...], w3g, precision=_HI,
                     preferred_element_type=jnp.float32) + b3_ref[...]
    out_ref[...] = (part_ref[...].astype(jnp.float32)
                    + ap_add[:, None, :] + ue_add[None, :, :])


def kernel(ap_hid, ue_hid, ue2ap_hid, ap2ue_hid, W1, b1, W2, b2, W3, b3):
    n_ap, d = ap_hid.shape
    n_ue = ue_hid.shape[0]
    e1 = ue2ap_hid.reshape(n_ap, n_ue, d)
    e2 = ap2ue_hid.reshape(n_ap, n_ue, d)
    b1r = b1.reshape(1, d)
    b2r = b2.reshape(1, d)
    b3r = b3.reshape(1, d)

    BU = 128
    NU = n_ue // BU

    full2 = lambda shape: pl.BlockSpec(shape, lambda j: (0, 0))
    ublk3 = pl.BlockSpec((n_ap, BU, d), lambda j: (0, j, 0))
    ublk2 = pl.BlockSpec((BU, d), lambda j: (j, 0))

    part, ue_sum, ap_sum = pl.pallas_call(
        functools.partial(_pass1_body, n_ap, d, BU, NU),
        grid=(NU,),
        in_specs=[
            full2((n_ap, d)),
            ublk2,
            ublk3,
            ublk3,
            full2((2 * d, d)),
            full2((1, d)),
            full2((2 * d, d)),
            full2((1, d)),
            full2((2 * d, d)),
        ],
        out_specs=[
            ublk3,
            ublk2,
            full2((n_ap, d)),
        ],
        out_shape=[
            jax.ShapeDtypeStruct((n_ap, n_ue, d), jnp.bfloat16),
            jax.ShapeDtypeStruct((n_ue, d), jnp.float32),
            jax.ShapeDtypeStruct((n_ap, d), jnp.float32),
        ],
    )(ap_hid, ue_hid, e1, e2, W1, b1r, W2, b2r, W3)

    BU2 = 128
    NU2 = n_ue // BU2
    ublk3b = pl.BlockSpec((n_ap, BU2, d), lambda j: (0, j, 0))
    ublk2b = pl.BlockSpec((BU2, d), lambda j: (j, 0))

    out = pl.pallas_call(
        functools.partial(_pass2_body, d),
        grid=(NU2,),
        in_specs=[
            ublk3b,
            full2((n_ap, d)),
            ublk2b,
            full2((2 * d, d)),
            full2((1, d)),
        ],
        out_specs=ublk3b,
        out_shape=jax.ShapeDtypeStruct((n_ap, n_ue, d), jnp.float32),
    )(part, ap_sum, ue_sum, W3, b3r)

    return out.reshape(n_ap * n_ue, d)


# speedup vs baseline: 2.6568x; 1.8670x over previous
import functools

import jax
import jax.numpy as jnp
from jax.experimental import pallas as pl
from jax.experimental.pallas import tpu as pltpu


_HI = jax.lax.Precision.HIGHEST
_DN0 = (((0,), (0,)), ((), ()))


def _pass1_body(n_ap, n_ue, d,
                e1t_ref, e2t_ref, ap_ref, ue_ref, w1_ref, b1_ref, w2_ref,
                b2_ref, w3_ref, b3_ref,
                partt_ref, ueaddt_ref, uesum_s, appre_s, uepre_s):
    j = pl.program_id(0)
    w1e = w1_ref[d:, :]
    w2e = w2_ref[d:, :]
    w3e = w3_ref[:d, :]
    w3g = w3_ref[d:, :]

    @pl.when(j == 0)
    def _init():
        appre_s[...] = jnp.dot(ap_ref[...], w1_ref[:d, :], precision=_HI,
                               preferred_element_type=jnp.float32) + b1_ref[...]
        uepre_s[...] = jnp.dot(ue_ref[...], w2_ref[:d, :], precision=_HI,
                               preferred_element_type=jnp.float32) + b2_ref[...]
        uesum_s[...] = jnp.zeros_like(uesum_s)

    e1t = e1t_ref[...]
    e2t = e2t_ref[...]
    t1 = jax.lax.dot_general(e1t, w1e, _DN0,
                             preferred_element_type=jnp.float32)
    t2 = jax.lax.dot_general(e2t, w2e, _DN0,
                             preferred_element_type=jnp.float32)
    r1 = jax.nn.relu(t1 + appre_s[pl.ds(j, 1), :])
    r2 = jax.nn.relu(t2 + uepre_s[...])
    s = r1 + r2
    uesum_s[...] += r2

    apsum_row = jnp.sum(r1, axis=0, keepdims=True)
    apg = jnp.dot(apsum_row, w3g, precision=_HI,
                  preferred_element_type=jnp.float32)
    part = (jax.lax.dot_general(e2t, w3e, _DN0,
                                preferred_element_type=jnp.float32)
            - jnp.dot(s, w3g, preferred_element_type=jnp.float32)
            + apg)
    partt_ref[...] = part.astype(jnp.bfloat16).T

    @pl.when(j == n_ap - 1)
    def _finish():
        ue_add = jnp.dot(uesum_s[...], w3g, precision=_HI,
                         preferred_element_type=jnp.float32) + b3_ref[...]
        ueaddt_ref[...] = ue_add.T


def _pass2_body(partt_ref, ueaddt_ref, outt_ref):
    outt_ref[...] = partt_ref[...].astype(jnp.float32) + ueaddt_ref[...]


def kernel(ap_hid, ue_hid, ue2ap_hid, ap2ue_hid, W1, b1, W2, b2, W3, b3):
    n_ap, d = ap_hid.shape
    n_ue = ue_hid.shape[0]
    E = n_ap * n_ue
    e1t = ue2ap_hid.T
    e2t = ap2ue_hid.T
    b1r = b1.reshape(1, d)
    b2r = b2.reshape(1, d)
    b3r = b3.reshape(1, d)

    full = lambda shape: pl.BlockSpec(shape, lambda j: (0,) * len(shape))
    ablk = pl.BlockSpec((d, n_ue), lambda j: (0, j))

    partt, ueaddt = pl.pallas_call(
        functools.partial(_pass1_body, n_ap, n_ue, d),
        grid=(n_ap,),
        in_specs=[
            ablk,
            ablk,
            full((n_ap, d)),
            full((n_ue, d)),
            full((2 * d, d)),
            full((1, d)),
            full((2 * d, d)),
            full((1, d)),
            full((2 * d, d)),
            full((1, d)),
        ],
        out_specs=[
            ablk,
            full((d, n_ue)),
        ],
        out_shape=[
            jax.ShapeDtypeStruct((d, E), jnp.bfloat16),
            jax.ShapeDtypeStruct((d, n_ue), jnp.float32),
        ],
        scratch_shapes=[
            pltpu.VMEM((n_ue, d), jnp.float32),
            pltpu.VMEM((n_ap, d), jnp.float32),
            pltpu.VMEM((n_ue, d), jnp.float32),
        ],
    )(e1t, e2t, ap_hid, ue_hid, W1, b1r, W2, b2r, W3, b3r)

    outt = pl.pallas_call(
        _pass2_body,
        grid=(n_ap,),
        in_specs=[ablk, full((d, n_ue))],
        out_specs=ablk,
        out_shape=jax.ShapeDtypeStruct((d, E), jnp.float32),
    )(partt, ueaddt)

    return outt.T


# speedup vs baseline: 2.6678x; 1.0042x over previous
import functools

import jax
import jax.numpy as jnp
from jax.experimental import pallas as pl
from jax.experimental.pallas import tpu as pltpu


_HI = jax.lax.Precision.HIGHEST
_DN0 = (((0,), (0,)), ((), ()))


def _pass1_body(n_ap, n_ue, d,
                e1t_ref, e2t_ref, ap_ref, ue_ref, w1_ref, b1_ref, w2_ref,
                b2_ref, w3_ref, b3_ref,
                partt_ref, ueaddt_ref, uesum_s, appre_s, uepre_s):
    j = pl.program_id(0)
    w1e = w1_ref[d:, :]
    w2e = w2_ref[d:, :]
    w3e = w3_ref[:d, :]
    w3g = w3_ref[d:, :]

    @pl.when(j == 0)
    def _init():
        appre_s[...] = jnp.dot(ap_ref[...], w1_ref[:d, :], precision=_HI,
                               preferred_element_type=jnp.float32) + b1_ref[...]
        uepre_s[...] = (jnp.dot(ue_ref[...], w2_ref[:d, :], precision=_HI,
                                preferred_element_type=jnp.float32)
                        + b2_ref[...]).astype(jnp.bfloat16)
        uesum_s[...] = jnp.zeros_like(uesum_s)

    e1t = e1t_ref[...]
    e2t = e2t_ref[...]
    t1 = jax.lax.dot_general(e1t, w1e, _DN0,
                             preferred_element_type=jnp.float32
                             ).astype(jnp.bfloat16)
    t2 = jax.lax.dot_general(e2t, w2e, _DN0,
                             preferred_element_type=jnp.float32
                             ).astype(jnp.bfloat16)
    r1 = jax.nn.relu(t1 + appre_s[pl.ds(j, 1), :].astype(jnp.bfloat16))
    r2 = jax.nn.relu(t2 + uepre_s[...])
    s = r1 + r2
    uesum_s[...] += r2

    apsum_row = jnp.sum(r1, axis=0, keepdims=True,
                        dtype=jnp.float32)
    apg = jnp.dot(apsum_row, w3g, precision=_HI,
                  preferred_element_type=jnp.float32)
    u3 = jax.lax.dot_general(e2t, w3e, _DN0,
                             preferred_element_type=jnp.float32
                             ).astype(jnp.bfloat16)
    sg = jnp.dot(s, w3g.astype(jnp.bfloat16),
                 preferred_element_type=jnp.float32).astype(jnp.bfloat16)
    part = u3 - sg + apg.astype(jnp.bfloat16)
    partt_ref[...] = part.T

    @pl.when(j == n_ap - 1)
    def _finish():
        ue_add = jnp.dot(uesum_s[...].astype(jnp.float32), w3g, precision=_HI,
                         preferred_element_type=jnp.float32) + b3_ref[...]
        ueaddt_ref[...] = ue_add.T


def _pass2_body(partt_ref, ueaddt_ref, outt_ref):
    outt_ref[...] = partt_ref[...].astype(jnp.float32) + ueaddt_ref[...]


def kernel(ap_hid, ue_hid, ue2ap_hid, ap2ue_hid, W1, b1, W2, b2, W3, b3):
    n_ap, d = ap_hid.shape
    n_ue = ue_hid.shape[0]
    E = n_ap * n_ue
    e1t = ue2ap_hid.T
    e2t = ap2ue_hid.T
    b1r = b1.reshape(1, d)
    b2r = b2.reshape(1, d)
    b3r = b3.reshape(1, d)

    full = lambda shape: pl.BlockSpec(shape, lambda j: (0,) * len(shape))
    ablk = pl.BlockSpec((d, n_ue), lambda j: (0, j))

    partt, ueaddt = pl.pallas_call(
        functools.partial(_pass1_body, n_ap, n_ue, d),
        grid=(n_ap,),
        in_specs=[
            ablk,
            ablk,
            full((n_ap, d)),
            full((n_ue, d)),
            full((2 * d, d)),
            full((1, d)),
            full((2 * d, d)),
            full((1, d)),
            full((2 * d, d)),
            full((1, d)),
        ],
        out_specs=[
            ablk,
            full((d, n_ue)),
        ],
        out_shape=[
            jax.ShapeDtypeStruct((d, E), jnp.bfloat16),
            jax.ShapeDtypeStruct((d, n_ue), jnp.float32),
        ],
        scratch_shapes=[
            pltpu.VMEM((n_ue, d), jnp.bfloat16),
            pltpu.VMEM((n_ap, d), jnp.float32),
            pltpu.VMEM((n_ue, d), jnp.bfloat16),
        ],
    )(e1t, e2t, ap_hid, ue_hid, W1, b1r, W2, b2r, W3, b3r)

    outt = pl.pallas_call(
        _pass2_body,
        grid=(n_ap,),
        in_specs=[ablk, full((d, n_ue))],
        out_specs=ablk,
        out_shape=jax.ShapeDtypeStruct((d, E), jnp.float32),
    )(partt, ueaddt)

    return outt.T
